# Initial kernel scaffold; baseline (speedup 1.0000x reference)
import jax
import jax.numpy as jnp
from jax.experimental import pallas as pl


def kernel(cat_feats, num_feats, atom_history, sem_history, cat_tables, num_w, num_b, mlp_W, mlp_b, ln_gamma, ln_beta, atom_table, sem_table):
    raise NotImplementedError("write your pallas kernel here")


# speedup vs baseline: 1.7618x; 1.7618x over previous
import functools

import jax
import jax.numpy as jnp
from jax import lax
from jax.experimental import pallas as pl
from jax.experimental.pallas import tpu as pltpu
from jax.experimental.pallas import tpu_sc as plsc

_B = 4096
_D = 32
_NCAT = 26
_NNUM = 13
_CATV = 100000
_LA = 200
_LS = 50

_NC = 2
_NS = 16
_NW = _NC * _NS

_CAT_K, _CAT_NT = 13, 2
_ATOM_K, _ATOM_NT = 20, 10
_SEM_K, _SEM_NT = 10, 5
_MAXK = 20


def _sc_gather_body(cat_tab, cat_idx, atom_tab, atom_idx, sem_tab, sem_idx,
                    cat_out, atom_out, sem_out, rows_v, idx_v, dsem):
    w = lax.axis_index("s") * _NC + lax.axis_index("c")
    specs = (
        (cat_tab, cat_idx, cat_out, _CAT_K, _CAT_NT),
        (atom_tab, atom_idx, atom_out, _ATOM_K, _ATOM_NT),
        (sem_tab, sem_idx, sem_out, _SEM_K, _SEM_NT),
    )
    for tab, idx2, out, K, nt in specs:
        rows128_pw = K * nt

        def tile(t, carry, tab=tab, idx2=idx2, out=out, K=K,
                 rows128_pw=rows128_pw):
            r0 = w * rows128_pw + t * K
            pltpu.sync_copy(idx2.at[pl.ds(r0 * 128, K * 128)],
                            idx_v.at[pl.ds(0, K * 128)])
            descs = [
                pltpu.async_copy(tab.at[idx_v.at[pl.ds(j * 128, 128)]],
                                 rows_v.at[pl.ds(j * 128, 128)], dsem)
                for j in range(K)
            ]
            for d in descs:
                d.wait()
            pltpu.sync_copy(rows_v.at[pl.ds(0, K * 128)],
                            out.at[pl.ds(r0 * 128, K * 128)])
            return carry

        lax.fori_loop(0, nt, tile, 0)


def _sc_gather_all(cat_tab, cat_idx2, atom_tab, atom_idx2, sem_tab, sem_idx2):
    kfn = pl.kernel(
        _sc_gather_body,
        out_type=(
            jax.ShapeDtypeStruct((_B * _NCAT, _D), jnp.float32),
            jax.ShapeDtypeStruct((_B * _LA, _D), jnp.float32),
            jax.ShapeDtypeStruct((_B * _LS, _D), jnp.float32),
        ),
        mesh=plsc.VectorSubcoreMesh(core_axis_name="c", subcore_axis_name="s",
                                    num_cores=_NC, num_subcores=_NS),
        scratch_types=[
            pltpu.VMEM((_MAXK * 128, _D), jnp.float32),
            pltpu.VMEM((_MAXK * 128,), jnp.int32),
            pltpu.SemaphoreType.DMA,
        ],
        compiler_params=pltpu.CompilerParams(use_tc_tiling_on_sc=False),
    )
    return kfn(cat_tab, cat_idx2, atom_tab, atom_idx2, sem_tab, sem_idx2)


def _gelu_exact(x):
    return 0.5 * x * (1.0 + lax.erf(x * 0.7071067811865476))


def _layernorm_last(x, g, b, eps=1e-5):
    mu = jnp.mean(x, axis=-1, keepdims=True)
    var = jnp.mean((x - mu) ** 2, axis=-1, keepdims=True)
    return (x - mu) * lax.rsqrt(var + eps) * g + b


def _mlp_body(cat_ref, nf_ref, nw_ref, nb_ref, W_ref, b_ref, g_ref, be_ref,
              fc_ref, fn_ref):
    W = W_ref[...]
    b = b_ref[...]
    g = g_ref[...]
    be = be_ref[...]
    h = jnp.dot(cat_ref[...], W, preferred_element_type=jnp.float32) + b
    fc_ref[...] = _layernorm_last(_gelu_exact(h), g, be)
    A = jnp.dot(nw_ref[...], W, preferred_element_type=jnp.float32)
    C = jnp.dot(nb_ref[...], W, preferred_element_type=jnp.float32) + b
    f = nf_ref[...]
    hn = f[:, :, None] * A[None, :, :] + C[None, :, :]
    fn_ref[...] = _layernorm_last(_gelu_exact(hn), g[None], be[None])


def _mlp_tc(cat_emb2, num_feats, num_w, num_b, mlp_W, mlp_b2, ln_g2, ln_b2):
    BB = 512
    CB = BB * _NCAT
    grid = (_B // BB,)
    return pl.pallas_call(
        _mlp_body,
        grid=grid,
        in_specs=[
            pl.BlockSpec((CB, _D), lambda i: (i, 0)),
            pl.BlockSpec((BB, _NNUM), lambda i: (i, 0)),
            pl.BlockSpec((_NNUM, _D), lambda i: (0, 0)),
            pl.BlockSpec((_NNUM, _D), lambda i: (0, 0)),
            pl.BlockSpec((_D, _D), lambda i: (0, 0)),
            pl.BlockSpec((1, _D), lambda i: (0, 0)),
            pl.BlockSpec((1, _D), lambda i: (0, 0)),
            pl.BlockSpec((1, _D), lambda i: (0, 0)),
        ],
        out_specs=[
            pl.BlockSpec((CB, _D), lambda i: (i, 0)),
            pl.BlockSpec((BB, _NNUM, _D), lambda i: (i, 0, 0)),
        ],
        out_shape=[
            jax.ShapeDtypeStruct((_B * _NCAT, _D), jnp.float32),
            jax.ShapeDtypeStruct((_B, _NNUM, _D), jnp.float32),
        ],
    )(cat_emb2, num_feats, num_w, num_b, mlp_W, mlp_b2, ln_g2, ln_b2)


def kernel(cat_feats, num_feats, atom_history, sem_history, cat_tables, num_w,
           num_b, mlp_W, mlp_b, ln_gamma, ln_beta, atom_table, sem_table):
    cat_tab2 = cat_tables.reshape(_NCAT * _CATV, _D)
    cat_idx = (cat_feats
               + jnp.arange(_NCAT, dtype=jnp.int32)[None, :] * _CATV)
    cat_idx2 = cat_idx.reshape(_B * _NCAT)
    atom_idx2 = atom_history.reshape(_B * _LA)
    sem_idx2 = sem_history.reshape(_B * _LS)

    cat_emb2, atom_tok2, sem_tok2 = _sc_gather_all(
        cat_tab2, cat_idx2, atom_table, atom_idx2, sem_table, sem_idx2)

    feat_cat2, feat_num = _mlp_tc(
        cat_emb2, num_feats, num_w, num_b, mlp_W,
        mlp_b.reshape(1, _D), ln_gamma.reshape(1, _D), ln_beta.reshape(1, _D))

    return jnp.concatenate([
        feat_cat2.reshape(_B, _NCAT, _D),
        feat_num,
        atom_tok2.reshape(_B, _LA, _D),
        sem_tok2.reshape(_B, _LS, _D),
    ], axis=1)
